# baseline (device time: 81972 ns/iter reference)
import jax
import jax.numpy as jnp
from jax import lax
from jax.experimental import pallas as pl
from jax.experimental.pallas import tpu as pltpu

N_DEV = 4


def kernel(x, w_mat):
    m, _ = x.shape
    _, n = w_mat.shape

    def body(x_ref, w_ref, out_ref, comm_ref, send_sems, recv_sems):
        my = lax.axis_index("i")
        left = lax.rem(my + (N_DEV - 1), N_DEV)
        right = lax.rem(my + 1, N_DEV)

        barrier_sem = pltpu.get_barrier_semaphore()
        for nbr in (left, right):
            pl.semaphore_signal(
                barrier_sem, inc=1,
                device_id=(nbr,), device_id_type=pl.DeviceIdType.MESH,
            )
        pl.semaphore_wait(barrier_sem, 2)

        partial = jnp.dot(
            x_ref[:, :], w_ref[:, :], preferred_element_type=jnp.float32
        )
        out_ref[:, :] = partial
        comm_ref[0, :, :] = partial.astype(jnp.bfloat16)

        for h in range(N_DEV - 1):
            rdma = pltpu.make_async_remote_copy(
                src_ref=comm_ref.at[h],
                dst_ref=comm_ref.at[h + 1],
                send_sem=send_sems.at[h],
                recv_sem=recv_sems.at[h + 1],
                device_id=(right,),
                device_id_type=pl.DeviceIdType.MESH,
            )
            rdma.start()
            rdma.wait()
            out_ref[:, :] = out_ref[:, :] + comm_ref[h + 1, :, :].astype(
                jnp.float32
            )

    return pl.pallas_call(
        body,
        out_shape=jax.ShapeDtypeStruct((m, n), jnp.float32),
        in_specs=[
            pl.BlockSpec(memory_space=pltpu.VMEM),
            pl.BlockSpec(memory_space=pltpu.VMEM),
        ],
        out_specs=pl.BlockSpec(memory_space=pltpu.VMEM),
        scratch_shapes=[
            pltpu.VMEM((N_DEV, m, n), jnp.bfloat16),
            pltpu.SemaphoreType.DMA((N_DEV,)),
            pltpu.SemaphoreType.DMA((N_DEV,)),
        ],
        compiler_params=pltpu.CompilerParams(collective_id=0),
    )(x, w_mat)


# device time: 35831 ns/iter; 2.2877x vs baseline; 2.2877x over previous
import jax
import jax.numpy as jnp
from jax import lax
from jax.experimental import pallas as pl
from jax.experimental.pallas import tpu as pltpu

N_DEV = 4


def kernel(x, w_mat):
    m, _ = x.shape
    _, n = w_mat.shape
    mc = m // N_DEV
    nh = n // 2
    n_steps = N_DEV - 1

    def body(x_ref, w_ref, out_ref, send_cw, send_ccw, recv_cw, recv_ccw,
             send_sems, recv_sems):
        my = lax.axis_index("i")
        left = (my + (N_DEV - 1)) % N_DEV
        right = (my + 1) % N_DEV

        barrier_sem = pltpu.get_barrier_semaphore()
        for nbr in (left, right):
            pl.semaphore_signal(
                barrier_sem, inc=1,
                device_id=(nbr,), device_id_type=pl.DeviceIdType.MESH,
            )
        pl.semaphore_wait(barrier_sem, 2)

        out_ref[:, :] = jnp.dot(
            x_ref[:, :], w_ref[:, :], preferred_element_type=jnp.float32
        )

        def rows(c):
            return pl.ds(c * mc, mc)

        cw_cols = pl.ds(0, nh)
        ccw_cols = pl.ds(nh, nh)

        def start_pair(src_cw, src_ccw, slot):
            cw = pltpu.make_async_remote_copy(
                src_ref=src_cw, dst_ref=recv_cw.at[slot],
                send_sem=send_sems.at[0, slot], recv_sem=recv_sems.at[0, slot],
                device_id=(right,), device_id_type=pl.DeviceIdType.MESH,
            )
            ccw = pltpu.make_async_remote_copy(
                src_ref=src_ccw, dst_ref=recv_ccw.at[slot],
                send_sem=send_sems.at[1, slot], recv_sem=recv_sems.at[1, slot],
                device_id=(left,), device_id_type=pl.DeviceIdType.MESH,
            )
            cw.start()
            ccw.start()
            return cw, ccw

        for s in range(n_steps):
            sc_cw = (my - s + N_DEV) % N_DEV
            rc_cw = (my - s - 1 + N_DEV) % N_DEV
            sc_ccw = (my + s) % N_DEV
            rc_ccw = (my + s + 1) % N_DEV

            send_cw[:, :] = out_ref[rows(sc_cw), cw_cols].astype(jnp.bfloat16)
            send_ccw[:, :] = out_ref[rows(sc_ccw), ccw_cols].astype(jnp.bfloat16)
            cw, ccw = start_pair(send_cw, send_ccw, s)
            cw.wait()
            ccw.wait()
            out_ref[rows(rc_cw), cw_cols] += recv_cw[s].astype(jnp.float32)
            out_ref[rows(rc_ccw), ccw_cols] += recv_ccw[s].astype(jnp.float32)

        for s in range(n_steps):
            j = n_steps + s
            rc_cw = (my - s + N_DEV) % N_DEV
            rc_ccw = (my + s) % N_DEV

            if s == 0:
                sc_cw = (my + 1) % N_DEV
                sc_ccw = (my + 3) % N_DEV
                send_cw[:, :] = out_ref[rows(sc_cw), cw_cols].astype(jnp.bfloat16)
                send_ccw[:, :] = out_ref[rows(sc_ccw), ccw_cols].astype(jnp.bfloat16)
                src_cw, src_ccw = send_cw, send_ccw
            else:
                src_cw, src_ccw = recv_cw.at[j - 1], recv_ccw.at[j - 1]
            cw, ccw = start_pair(src_cw, src_ccw, j)
            cw.wait()
            ccw.wait()
            out_ref[rows(rc_cw), cw_cols] = recv_cw[j].astype(jnp.float32)
            out_ref[rows(rc_ccw), ccw_cols] = recv_ccw[j].astype(jnp.float32)

    return pl.pallas_call(
        body,
        out_shape=jax.ShapeDtypeStruct((m, n), jnp.float32),
        in_specs=[
            pl.BlockSpec(memory_space=pltpu.VMEM),
            pl.BlockSpec(memory_space=pltpu.VMEM),
        ],
        out_specs=pl.BlockSpec(memory_space=pltpu.VMEM),
        scratch_shapes=[
            pltpu.VMEM((mc, nh), jnp.bfloat16),
            pltpu.VMEM((mc, nh), jnp.bfloat16),
            pltpu.VMEM((2 * n_steps, mc, nh), jnp.bfloat16),
            pltpu.VMEM((2 * n_steps, mc, nh), jnp.bfloat16),
            pltpu.SemaphoreType.DMA((2, 2 * n_steps)),
            pltpu.SemaphoreType.DMA((2, 2 * n_steps)),
        ],
        compiler_params=pltpu.CompilerParams(collective_id=0),
    )(x, w_mat)


# device time: 35445 ns/iter; 2.3127x vs baseline; 1.0109x over previous
import jax
import jax.numpy as jnp
from jax import lax
from jax.experimental import pallas as pl
from jax.experimental.pallas import tpu as pltpu

N_DEV = 4
N_HOPS = 2 * (N_DEV - 1)


def kernel(x, w_mat):
    m, _ = x.shape
    _, n = w_mat.shape
    mc = m // N_DEV
    nh = n // 2

    def body(x_ref, w_ref, out_ref, stage_cw, stage_ccw, recv_cw, recv_ccw,
             send_sems, recv_sems):
        my = lax.axis_index("i")
        left = (my + (N_DEV - 1)) % N_DEV
        right = (my + 1) % N_DEV

        barrier_sem = pltpu.get_barrier_semaphore()
        for nbr in (left, right):
            pl.semaphore_signal(
                barrier_sem, inc=1,
                device_id=(nbr,), device_id_type=pl.DeviceIdType.MESH,
            )
        pl.semaphore_wait(barrier_sem, 2)

        out_ref[:, :] = jnp.dot(
            x_ref[:, :], w_ref[:, :], preferred_element_type=jnp.float32
        )

        def rows(c):
            return pl.ds((c % N_DEV) * mc, mc)

        cw_cols = pl.ds(0, nh)
        ccw_cols = pl.ds(nh, nh)

        def mk(dir_idx, h, src):
            recv = recv_cw if dir_idx == 0 else recv_ccw
            tgt = right if dir_idx == 0 else left
            return pltpu.make_async_remote_copy(
                src_ref=src, dst_ref=recv.at[h],
                send_sem=send_sems.at[dir_idx, h],
                recv_sem=recv_sems.at[dir_idx, h],
                device_id=(tgt,), device_id_type=pl.DeviceIdType.MESH,
            )

        rc_cw = [my - 1, my - 2, my - 3, my, my - 1, my - 2]
        rc_ccw = [my + 1, my + 2, my + 3, my, my + 1, my + 2]

        descs_cw = [None] * N_HOPS
        descs_ccw = [None] * N_HOPS

        stage_cw[0] = out_ref[rows(my), cw_cols].astype(jnp.bfloat16)
        stage_ccw[0] = out_ref[rows(my), ccw_cols].astype(jnp.bfloat16)
        descs_cw[0] = mk(0, 0, stage_cw.at[0])
        descs_ccw[0] = mk(1, 0, stage_ccw.at[0])
        descs_cw[0].start()
        descs_ccw[0].start()

        for h in range(N_DEV - 1):
            k = (h + 1) % 2
            descs_cw[h].wait_recv()
            acc = out_ref[rows(rc_cw[h]), cw_cols] + recv_cw[h].astype(
                jnp.float32
            )
            if h >= 1:
                descs_cw[h - 1].wait_send()
            stage_cw[k] = acc.astype(jnp.bfloat16)
            descs_cw[h + 1] = mk(0, h + 1, stage_cw.at[k])
            descs_cw[h + 1].start()
            out_ref[rows(rc_cw[h]), cw_cols] = acc

            descs_ccw[h].wait_recv()
            acc = out_ref[rows(rc_ccw[h]), ccw_cols] + recv_ccw[h].astype(
                jnp.float32
            )
            if h >= 1:
                descs_ccw[h - 1].wait_send()
            stage_ccw[k] = acc.astype(jnp.bfloat16)
            descs_ccw[h + 1] = mk(1, h + 1, stage_ccw.at[k])
            descs_ccw[h + 1].start()
            out_ref[rows(rc_ccw[h]), ccw_cols] = acc

        for h in range(N_DEV - 1, N_HOPS):
            descs_cw[h].wait_recv()
            if h + 1 < N_HOPS:
                descs_cw[h + 1] = mk(0, h + 1, recv_cw.at[h])
                descs_cw[h + 1].start()
            out_ref[rows(rc_cw[h]), cw_cols] = recv_cw[h].astype(jnp.float32)

            descs_ccw[h].wait_recv()
            if h + 1 < N_HOPS:
                descs_ccw[h + 1] = mk(1, h + 1, recv_ccw.at[h])
                descs_ccw[h + 1].start()
            out_ref[rows(rc_ccw[h]), ccw_cols] = recv_ccw[h].astype(
                jnp.float32
            )

        for h in range(2, N_HOPS):
            descs_cw[h].wait_send()
            descs_ccw[h].wait_send()

    return pl.pallas_call(
        body,
        out_shape=jax.ShapeDtypeStruct((m, n), jnp.float32),
        in_specs=[
            pl.BlockSpec(memory_space=pltpu.VMEM),
            pl.BlockSpec(memory_space=pltpu.VMEM),
        ],
        out_specs=pl.BlockSpec(memory_space=pltpu.VMEM),
        scratch_shapes=[
            pltpu.VMEM((2, mc, nh), jnp.bfloat16),
            pltpu.VMEM((2, mc, nh), jnp.bfloat16),
            pltpu.VMEM((N_HOPS, mc, nh), jnp.bfloat16),
            pltpu.VMEM((N_HOPS, mc, nh), jnp.bfloat16),
            pltpu.SemaphoreType.DMA((2, N_HOPS)),
            pltpu.SemaphoreType.DMA((2, N_HOPS)),
        ],
        compiler_params=pltpu.CompilerParams(collective_id=0),
    )(x, w_mat)


# device time: 34887 ns/iter; 2.3496x vs baseline; 1.0160x over previous
import jax
import jax.numpy as jnp
from jax import lax
from jax.experimental import pallas as pl
from jax.experimental.pallas import tpu as pltpu

N_DEV = 4
N_HOPS = 2 * (N_DEV - 1)


def kernel(x, w_mat):
    m, _ = x.shape
    _, n = w_mat.shape
    mc = m // N_DEV
    nh = n // 2

    def body(x_ref, w_ref, out_ref, stage_cw, stage_ccw, recv_cw, recv_ccw,
             send_sems, recv_sems):
        my = lax.axis_index("i")
        left = (my + (N_DEV - 1)) % N_DEV
        right = (my + 1) % N_DEV

        barrier_sem = pltpu.get_barrier_semaphore()
        for nbr in (left, right):
            pl.semaphore_signal(
                barrier_sem, inc=1,
                device_id=(nbr,), device_id_type=pl.DeviceIdType.MESH,
            )
        pl.semaphore_wait(barrier_sem, 2)

        def rows(c):
            return pl.ds((c % N_DEV) * mc, mc)

        out_ref[rows(my), :] = jnp.dot(
            x_ref[rows(my), :], w_ref[:, :], preferred_element_type=jnp.float32
        )

        cw_cols = pl.ds(0, nh)
        ccw_cols = pl.ds(nh, nh)

        def mk(dir_idx, h, src):
            recv = recv_cw if dir_idx == 0 else recv_ccw
            tgt = right if dir_idx == 0 else left
            return pltpu.make_async_remote_copy(
                src_ref=src, dst_ref=recv.at[h],
                send_sem=send_sems.at[dir_idx, h],
                recv_sem=recv_sems.at[dir_idx, h],
                device_id=(tgt,), device_id_type=pl.DeviceIdType.MESH,
            )

        rc_cw = [my - 1, my - 2, my - 3, my, my - 1, my - 2]
        rc_ccw = [my + 1, my + 2, my + 3, my, my + 1, my + 2]

        descs_cw = [None] * N_HOPS
        descs_ccw = [None] * N_HOPS

        stage_cw[0] = out_ref[rows(my), cw_cols].astype(jnp.bfloat16)
        stage_ccw[0] = out_ref[rows(my), ccw_cols].astype(jnp.bfloat16)
        descs_cw[0] = mk(0, 0, stage_cw.at[0])
        descs_ccw[0] = mk(1, 0, stage_ccw.at[0])
        descs_cw[0].start()
        descs_ccw[0].start()

        for j in range(1, N_DEV):
            out_ref[rows(my + j), :] = jnp.dot(
                x_ref[rows(my + j), :], w_ref[:, :],
                preferred_element_type=jnp.float32,
            )

        for h in range(N_DEV - 1):
            k = (h + 1) % 2
            descs_cw[h].wait_recv()
            acc = out_ref[rows(rc_cw[h]), cw_cols] + recv_cw[h].astype(
                jnp.float32
            )
            if h >= 1:
                descs_cw[h - 1].wait_send()
            stage_cw[k] = acc.astype(jnp.bfloat16)
            descs_cw[h + 1] = mk(0, h + 1, stage_cw.at[k])
            descs_cw[h + 1].start()

            descs_ccw[h].wait_recv()
            acc_ccw = out_ref[rows(rc_ccw[h]), ccw_cols] + recv_ccw[h].astype(
                jnp.float32
            )
            if h >= 1:
                descs_ccw[h - 1].wait_send()
            stage_ccw[k] = acc_ccw.astype(jnp.bfloat16)
            descs_ccw[h + 1] = mk(1, h + 1, stage_ccw.at[k])
            descs_ccw[h + 1].start()
            if h == N_DEV - 2:
                out_ref[rows(rc_cw[h]), cw_cols] = acc
                out_ref[rows(rc_ccw[h]), ccw_cols] = acc_ccw

        for h in range(N_DEV - 1, N_HOPS):
            descs_cw[h].wait_recv()
            if h + 1 < N_HOPS:
                descs_cw[h + 1] = mk(0, h + 1, recv_cw.at[h])
                descs_cw[h + 1].start()
            out_ref[rows(rc_cw[h]), cw_cols] = recv_cw[h].astype(jnp.float32)

            descs_ccw[h].wait_recv()
            if h + 1 < N_HOPS:
                descs_ccw[h + 1] = mk(1, h + 1, recv_ccw.at[h])
                descs_ccw[h + 1].start()
            out_ref[rows(rc_ccw[h]), ccw_cols] = recv_ccw[h].astype(
                jnp.float32
            )

        for h in range(2, N_HOPS):
            descs_cw[h].wait_send()
            descs_ccw[h].wait_send()

    return pl.pallas_call(
        body,
        out_shape=jax.ShapeDtypeStruct((m, n), jnp.float32),
        in_specs=[
            pl.BlockSpec(memory_space=pltpu.VMEM),
            pl.BlockSpec(memory_space=pltpu.VMEM),
        ],
        out_specs=pl.BlockSpec(memory_space=pltpu.VMEM),
        scratch_shapes=[
            pltpu.VMEM((2, mc, nh), jnp.bfloat16),
            pltpu.VMEM((2, mc, nh), jnp.bfloat16),
            pltpu.VMEM((N_HOPS, mc, nh), jnp.bfloat16),
            pltpu.VMEM((N_HOPS, mc, nh), jnp.bfloat16),
            pltpu.SemaphoreType.DMA((2, N_HOPS)),
            pltpu.SemaphoreType.DMA((2, N_HOPS)),
        ],
        compiler_params=pltpu.CompilerParams(collective_id=0),
    )(x, w_mat)


# device time: 28147 ns/iter; 2.9123x vs baseline; 1.2395x over previous
import jax
import jax.numpy as jnp
from jax import lax
from jax.experimental import pallas as pl
from jax.experimental.pallas import tpu as pltpu

N_DEV = 4
N_HOPS = 2 * (N_DEV - 1)
STRIPS = 2
N_RINGS = 2 * STRIPS


def kernel(x, w_mat):
    m, _ = x.shape
    _, n = w_mat.shape
    mc = m // N_DEV
    nq = n // N_RINGS

    def body(x_ref, w_ref, out_ref, stage, recv, send_sems, recv_sems):
        my = lax.axis_index("i")
        left = (my + (N_DEV - 1)) % N_DEV
        right = (my + 1) % N_DEV

        barrier_sem = pltpu.get_barrier_semaphore()
        for nbr in (left, right):
            pl.semaphore_signal(
                barrier_sem, inc=1,
                device_id=(nbr,), device_id_type=pl.DeviceIdType.MESH,
            )
        pl.semaphore_wait(barrier_sem, 2)

        def rows(c):
            return pl.ds((c % N_DEV) * mc, mc)

        ring_dir = [r % 2 for r in range(N_RINGS)]
        ring_cols = [
            pl.ds((r % 2) * (n // 2) + (r // 2) * nq, nq)
            for r in range(N_RINGS)
        ]
        rc_tab = [
            [my - 1, my - 2, my - 3, my, my - 1, my - 2],
            [my + 1, my + 2, my + 3, my, my + 1, my + 2],
        ]

        def mk(r, h, src):
            return pltpu.make_async_remote_copy(
                src_ref=src, dst_ref=recv.at[r, h],
                send_sem=send_sems.at[r, h],
                recv_sem=recv_sems.at[r, h],
                device_id=(right if ring_dir[r] == 0 else left,),
                device_id_type=pl.DeviceIdType.MESH,
            )

        out_ref[rows(my), :] = jnp.dot(
            x_ref[rows(my), :], w_ref[:, :], preferred_element_type=jnp.float32
        )

        descs = [[None] * N_HOPS for _ in range(N_RINGS)]
        for r in range(N_RINGS):
            stage[r, 0] = out_ref[rows(my), ring_cols[r]].astype(jnp.bfloat16)
            descs[r][0] = mk(r, 0, stage.at[r, 0])
            descs[r][0].start()

        for j in range(1, N_DEV):
            out_ref[rows(my + j), :] = jnp.dot(
                x_ref[rows(my + j), :], w_ref[:, :],
                preferred_element_type=jnp.float32,
            )

        for h in range(N_DEV - 1):
            k = (h + 1) % 2
            own_accs = []
            for r in range(N_RINGS):
                rc = rc_tab[ring_dir[r]][h]
                descs[r][h].wait_recv()
                acc = out_ref[rows(rc), ring_cols[r]] + recv[r, h].astype(
                    jnp.float32
                )
                if h >= 1:
                    descs[r][h - 1].wait_send()
                stage[r, k] = acc.astype(jnp.bfloat16)
                descs[r][h + 1] = mk(r, h + 1, stage.at[r, k])
                descs[r][h + 1].start()
                if h == N_DEV - 2:
                    own_accs.append((rc, r, acc))
            for rc, r, acc in own_accs:
                out_ref[rows(rc), ring_cols[r]] = acc

        for h in range(N_DEV - 1, N_HOPS):
            stores = []
            for r in range(N_RINGS):
                descs[r][h].wait_recv()
                if h + 1 < N_HOPS:
                    descs[r][h + 1] = mk(r, h + 1, recv.at[r, h])
                    descs[r][h + 1].start()
                stores.append(r)
            for r in stores:
                rc = rc_tab[ring_dir[r]][h]
                out_ref[rows(rc), ring_cols[r]] = recv[r, h].astype(
                    jnp.float32
                )

        for r in range(N_RINGS):
            for h in range(2, N_HOPS):
                descs[r][h].wait_send()

    return pl.pallas_call(
        body,
        out_shape=jax.ShapeDtypeStruct((m, n), jnp.float32),
        in_specs=[
            pl.BlockSpec(memory_space=pltpu.VMEM),
            pl.BlockSpec(memory_space=pltpu.VMEM),
        ],
        out_specs=pl.BlockSpec(memory_space=pltpu.VMEM),
        scratch_shapes=[
            pltpu.VMEM((N_RINGS, 2, mc, nq), jnp.bfloat16),
            pltpu.VMEM((N_RINGS, N_HOPS, mc, nq), jnp.bfloat16),
            pltpu.SemaphoreType.DMA((N_RINGS, N_HOPS)),
            pltpu.SemaphoreType.DMA((N_RINGS, N_HOPS)),
        ],
        compiler_params=pltpu.CompilerParams(collective_id=0),
    )(x, w_mat)


# device time: 27443 ns/iter; 2.9870x vs baseline; 1.0257x over previous
import jax
import jax.numpy as jnp
from jax import lax
from jax.experimental import pallas as pl
from jax.experimental.pallas import tpu as pltpu

N_DEV = 4
N_HOPS = 2 * (N_DEV - 1)
STRIPS = 4
N_RINGS = 2 * STRIPS


def kernel(x, w_mat):
    m, _ = x.shape
    _, n = w_mat.shape
    mc = m // N_DEV
    nq = n // N_RINGS

    def body(x_ref, w_ref, out_ref, stage, recv, send_sems, recv_sems):
        my = lax.axis_index("i")
        left = (my + (N_DEV - 1)) % N_DEV
        right = (my + 1) % N_DEV

        barrier_sem = pltpu.get_barrier_semaphore()
        for nbr in (left, right):
            pl.semaphore_signal(
                barrier_sem, inc=1,
                device_id=(nbr,), device_id_type=pl.DeviceIdType.MESH,
            )
        pl.semaphore_wait(barrier_sem, 2)

        def rows(c):
            return pl.ds((c % N_DEV) * mc, mc)

        ring_dir = [r % 2 for r in range(N_RINGS)]
        ring_cols = [
            pl.ds((r % 2) * (n // 2) + (r // 2) * nq, nq)
            for r in range(N_RINGS)
        ]
        rc_tab = [
            [my - 1, my - 2, my - 3, my, my - 1, my - 2],
            [my + 1, my + 2, my + 3, my, my + 1, my + 2],
        ]

        def mk(r, h, src):
            return pltpu.make_async_remote_copy(
                src_ref=src, dst_ref=recv.at[r, h],
                send_sem=send_sems.at[r, h],
                recv_sem=recv_sems.at[r, h],
                device_id=(right if ring_dir[r] == 0 else left,),
                device_id_type=pl.DeviceIdType.MESH,
            )

        out_ref[rows(my), :] = jnp.dot(
            x_ref[rows(my), :], w_ref[:, :], preferred_element_type=jnp.float32
        )

        descs = [[None] * N_HOPS for _ in range(N_RINGS)]
        for r in range(N_RINGS):
            stage[r, 0] = out_ref[rows(my), ring_cols[r]].astype(jnp.bfloat16)
            descs[r][0] = mk(r, 0, stage.at[r, 0])
            descs[r][0].start()

        for j in range(1, N_DEV):
            out_ref[rows(my + j), :] = jnp.dot(
                x_ref[rows(my + j), :], w_ref[:, :],
                preferred_element_type=jnp.float32,
            )

        for h in range(N_DEV - 1):
            k = (h + 1) % 2
            own_accs = []
            for r in range(N_RINGS):
                rc = rc_tab[ring_dir[r]][h]
                descs[r][h].wait_recv()
                acc = out_ref[rows(rc), ring_cols[r]] + recv[r, h].astype(
                    jnp.float32
                )
                if h >= 1:
                    descs[r][h - 1].wait_send()
                stage[r, k] = acc.astype(jnp.bfloat16)
                descs[r][h + 1] = mk(r, h + 1, stage.at[r, k])
                descs[r][h + 1].start()
                if h == N_DEV - 2:
                    own_accs.append((rc, r, acc))
            for rc, r, acc in own_accs:
                out_ref[rows(rc), ring_cols[r]] = acc

        for h in range(N_DEV - 1, N_HOPS):
            stores = []
            for r in range(N_RINGS):
                descs[r][h].wait_recv()
                if h + 1 < N_HOPS:
                    descs[r][h + 1] = mk(r, h + 1, recv.at[r, h])
                    descs[r][h + 1].start()
                stores.append(r)
            for r in stores:
                rc = rc_tab[ring_dir[r]][h]
                out_ref[rows(rc), ring_cols[r]] = recv[r, h].astype(
                    jnp.float32
                )

        for r in range(N_RINGS):
            for h in range(2, N_HOPS):
                descs[r][h].wait_send()

    return pl.pallas_call(
        body,
        out_shape=jax.ShapeDtypeStruct((m, n), jnp.float32),
        in_specs=[
            pl.BlockSpec(memory_space=pltpu.VMEM),
            pl.BlockSpec(memory_space=pltpu.VMEM),
        ],
        out_specs=pl.BlockSpec(memory_space=pltpu.VMEM),
        scratch_shapes=[
            pltpu.VMEM((N_RINGS, 2, mc, nq), jnp.bfloat16),
            pltpu.VMEM((N_RINGS, N_HOPS, mc, nq), jnp.bfloat16),
            pltpu.SemaphoreType.DMA((N_RINGS, N_HOPS)),
            pltpu.SemaphoreType.DMA((N_RINGS, N_HOPS)),
        ],
        compiler_params=pltpu.CompilerParams(collective_id=0),
    )(x, w_mat)
